# Initial kernel scaffold; baseline (speedup 1.0000x reference)
import jax
import jax.numpy as jnp
from jax.experimental import pallas as pl


def kernel(X, word_table, pos_table):
    raise NotImplementedError("write your pallas kernel here")


# speedup vs baseline: 5.0131x; 5.0131x over previous
import functools

import jax
import jax.numpy as jnp
from jax import lax
from jax.experimental import pallas as pl
from jax.experimental.pallas import tpu as pltpu
from jax.experimental.pallas import tpu_sc as plsc

VOCAB = 100000
MAX_LEN = 200
EMB = 128
BATCH = 4096
SEQ = 200

NUM_WORKERS = 32
TOK_PER_W = BATCH * SEQ // NUM_WORKERS
CHUNK = 40
CHUNKS_PER_SEQ = SEQ // CHUNK
NUM_CHUNKS = TOK_PER_W // CHUNK

_mesh = plsc.VectorSubcoreMesh(core_axis_name="c", subcore_axis_name="s")


@functools.partial(
    pl.kernel,
    mesh=_mesh,
    out_type=jax.ShapeDtypeStruct((BATCH * SEQ, EMB), jnp.float32),
    scratch_types=[
        pltpu.VMEM_SHARED((MAX_LEN, EMB), jnp.float32),
        pltpu.VMEM((CHUNK,), jnp.int32),
        pltpu.VMEM((CHUNK, EMB), jnp.float32),
        pltpu.SemaphoreType.DMA,
    ],
)
def _embed(x_hbm, wt_hbm, pos_hbm, out_hbm, pos_sh, idx_v, buf_v, sem):
    sid = lax.axis_index("s")
    wid = sid * 2 + lax.axis_index("c")
    base = wid * TOK_PER_W

    @pl.when(sid == 0)
    def _load_pos():
        pltpu.sync_copy(pos_hbm, pos_sh)

    plsc.subcore_barrier()

    @pl.loop(0, NUM_CHUNKS)
    def _chunk(g):
        tok0 = base + g * CHUNK
        pos0 = (g % CHUNKS_PER_SEQ) * CHUNK
        pltpu.sync_copy(x_hbm.at[pl.ds(tok0, CHUNK)], idx_v)
        pltpu.sync_copy(pos_sh.at[pl.ds(pos0, CHUNK)], buf_v)
        pltpu.async_copy(wt_hbm.at[idx_v], buf_v, sem, add=True).wait()
        pltpu.sync_copy(buf_v, out_hbm.at[pl.ds(tok0, CHUNK)])


def kernel(X, word_table, pos_table):
    out = _embed(X.reshape(-1), word_table, pos_table)
    return out.reshape(BATCH, SEQ, EMB)


# speedup vs baseline: 8.9292x; 1.7812x over previous
import jax
import jax.numpy as jnp
from jax import lax
from jax.experimental import pallas as pl
from jax.experimental.pallas import tpu as pltpu
from jax.experimental.pallas import tpu_sc as plsc

VOCAB = 100000
MAX_LEN = 200
EMB = 128
BATCH = 4096
SEQ = 200

NUM_WORKERS = 32
TOK_PER_W = BATCH * SEQ // NUM_WORKERS
CHUNK = 40
CHUNKS_PER_SEQ = SEQ // CHUNK
N = TOK_PER_W // CHUNK
NBUF = 4

_mesh = plsc.VectorSubcoreMesh(core_axis_name="c", subcore_axis_name="s")

_scratch = (
    [pltpu.VMEM_SHARED((MAX_LEN, EMB), jnp.float32)]
    + [pltpu.VMEM((CHUNK, EMB), jnp.float32) for _ in range(NBUF)]
    + [pltpu.VMEM((CHUNK,), jnp.int32) for _ in range(NBUF)]
    + [pltpu.SemaphoreType.DMA for _ in range(4 * NBUF)]
)


@jax.jit
def _embed_call(x, wt, pos):
    @pl.kernel(
        out_type=jax.ShapeDtypeStruct((BATCH * SEQ, EMB), jnp.float32),
        mesh=_mesh,
        scratch_types=_scratch,
    )
    def _embed(x_hbm, wt_hbm, pos_hbm, out_hbm, pos_sh, *scr):
        bufs = scr[0:NBUF]
        idxs = scr[NBUF:2 * NBUF]
        sem_init = scr[2 * NBUF:3 * NBUF]
        sem_idx = scr[3 * NBUF:4 * NBUF]
        sem_g = scr[4 * NBUF:5 * NBUF]
        sem_wb = scr[5 * NBUF:6 * NBUF]

        sid = lax.axis_index("s")
        wid = sid * 2 + lax.axis_index("c")
        base = wid * TOK_PER_W

        @pl.when(sid == 0)
        def _load_pos():
            pltpu.sync_copy(pos_hbm, pos_sh)

        plsc.subcore_barrier()

        def d_init(g, b):
            pos0 = (g % CHUNKS_PER_SEQ) * CHUNK
            return pltpu.make_async_copy(
                pos_sh.at[pl.ds(pos0, CHUNK)], bufs[b], sem_init[b])

        def d_idx(g, b):
            return pltpu.make_async_copy(
                x_hbm.at[pl.ds(base + g * CHUNK, CHUNK)], idxs[b], sem_idx[b])

        def d_gat(b):
            return pltpu.make_async_copy(wt_hbm.at[idxs[b]], bufs[b], sem_g[b])

        def d_wb(g, b):
            return pltpu.make_async_copy(
                bufs[b], out_hbm.at[pl.ds(base + g * CHUNK, CHUNK)], sem_wb[b])

        def issue_pre(g, b):
            d_init(g, b).start()
            d_idx(g, b).start()

        def issue_gather(g, b):
            d_init(g, b).wait()
            d_idx(g, b).wait()
            d_gat(b).start(add=True)

        def issue_wb(g, b):
            d_gat(b).wait()
            d_wb(g, b).start()

        def body(i, phase, drain):
            b0 = phase % NBUF
            b1 = (phase + 1) % NBUF
            if drain:
                d_wb(i - (NBUF - 1), b1).wait()
            issue_pre(i + 1, b1)
            issue_wb(i, b0)
            issue_gather(i + 1, b1)

        issue_pre(0, 0)
        issue_gather(0, 0)
        for i in range(NBUF - 1):
            body(i, i, drain=False)

        @pl.loop(NBUF - 1, N - 1, step=NBUF)
        def _steady(i0):
            for db in range(NBUF):
                body(i0 + db, NBUF - 1 + db, drain=True)

        issue_wb(N - 1, (N - 1) % NBUF)
        for g in range(N - NBUF, N):
            d_wb(g, g % NBUF).wait()

    return _embed(x, wt, pos)


def kernel(X, word_table, pos_table):
    out = _embed_call(X.reshape(-1), word_table, pos_table)
    return out.reshape(BATCH, SEQ, EMB)


# speedup vs baseline: 13.1476x; 1.4724x over previous
import jax
import jax.numpy as jnp
from jax import lax
from jax.experimental import pallas as pl
from jax.experimental.pallas import tpu as pltpu
from jax.experimental.pallas import tpu_sc as plsc

VOCAB = 100000
MAX_LEN = 200
EMB = 128
BATCH = 4096
SEQ = 200

NUM_WORKERS = 32
TOK_PER_W = BATCH * SEQ // NUM_WORKERS
CHUNK = 80
N = TOK_PER_W // CHUNK
NBUF = 5

_POS_PLAN = {
    0: [(0, 0, 80)],
    1: [(80, 0, 80)],
    2: [(160, 0, 40), (0, 40, 40)],
    3: [(40, 0, 80)],
    4: [(120, 0, 80)],
}

_mesh = plsc.VectorSubcoreMesh(core_axis_name="c", subcore_axis_name="s")

_scratch = (
    [pltpu.VMEM_SHARED((MAX_LEN, EMB), jnp.float32)]
    + [pltpu.VMEM((N, CHUNK), jnp.int32)]
    + [pltpu.VMEM((CHUNK, EMB), jnp.float32) for _ in range(NBUF)]
    + [pltpu.SemaphoreType.DMA for _ in range(3 * NBUF)]
)


@jax.jit
def _embed_call(x2d, wt, pos):
    @pl.kernel(
        out_type=jax.ShapeDtypeStruct((BATCH * SEQ, EMB), jnp.float32),
        mesh=_mesh,
        scratch_types=_scratch,
    )
    def _embed(x_hbm, wt_hbm, pos_hbm, out_hbm, pos_sh, idx2d, *scr):
        bufs = scr[0:NBUF]
        sem_init = scr[NBUF:2 * NBUF]
        sem_g = scr[2 * NBUF:3 * NBUF]
        sem_wb = scr[3 * NBUF:4 * NBUF]

        sid = lax.axis_index("s")
        wid = sid * 2 + lax.axis_index("c")
        base = wid * TOK_PER_W

        @pl.when(sid == 0)
        def _load_pos():
            pltpu.sync_copy(pos_hbm, pos_sh)

        plsc.subcore_barrier()

        pltpu.sync_copy(x_hbm.at[pl.ds(wid * N, N)], idx2d)

        def init_descs(v, b):
            return [
                pltpu.make_async_copy(
                    pos_sh.at[pl.ds(po, n)],
                    bufs[b].at[pl.ds(bo, n)],
                    sem_init[b])
                for po, bo, n in _POS_PLAN[v]
            ]

        def d_gat(g, b):
            return pltpu.make_async_copy(
                wt_hbm.at[idx2d.at[g]], bufs[b], sem_g[b])

        def d_wb(g, b):
            return pltpu.make_async_copy(
                bufs[b], out_hbm.at[pl.ds(base + g * CHUNK, CHUNK)],
                sem_wb[b])

        def issue_pre(v, b):
            for d in init_descs(v, b):
                d.start()

        def issue_gather(g, v, b):
            for d in init_descs(v, b):
                d.wait()
            d_gat(g, b).start(add=True)

        def issue_wb(g, b):
            d_gat(g, b).wait()
            d_wb(g, b).start()

        def body(i, phase, drain):
            b0 = phase % NBUF
            b1 = (phase + 1) % NBUF
            if drain:
                d_wb(i - (NBUF - 1), b1).wait()
            issue_pre(b1, b1)
            issue_wb(i, b0)
            issue_gather(i + 1, b1, b1)

        issue_pre(0, 0)
        issue_gather(0, 0, 0)
        for i in range(NBUF - 1):
            body(i, i, drain=False)

        @pl.loop(NBUF - 1, N - 1, step=NBUF)
        def _steady(i0):
            for db in range(NBUF):
                body(i0 + db, NBUF - 1 + db, drain=True)

        issue_wb(N - 1, (N - 1) % NBUF)
        for g in range(N - NBUF, N):
            d_wb(g, g % NBUF).wait()

    return _embed(x2d, wt, pos)


def kernel(X, word_table, pos_table):
    out = _embed_call(X.reshape(-1, CHUNK), word_table, pos_table)
    return out.reshape(BATCH, SEQ, EMB)


# speedup vs baseline: 18.7691x; 1.4276x over previous
import jax
import jax.numpy as jnp
from jax import lax
from jax.experimental import pallas as pl
from jax.experimental.pallas import tpu as pltpu
from jax.experimental.pallas import tpu_sc as plsc

VOCAB = 100000
MAX_LEN = 200
EMB = 128
BATCH = 4096
SEQ = 200

NUM_WORKERS = 32
TOK_PER_W = BATCH * SEQ // NUM_WORKERS
CHUNK = 80
N = TOK_PER_W // CHUNK
NBUF = 5

_POS_PLAN = {
    0: [(0, 0, 80)],
    1: [(80, 0, 80)],
    2: [(160, 0, 40), (0, 40, 40)],
    3: [(40, 0, 80)],
    4: [(120, 0, 80)],
}

_mesh = plsc.VectorSubcoreMesh(core_axis_name="c", subcore_axis_name="s")

_scratch = (
    [pltpu.VMEM_SHARED((MAX_LEN, EMB), jnp.float32)]
    + [pltpu.VMEM((N, CHUNK), jnp.int32)]
    + [pltpu.VMEM((CHUNK, EMB), jnp.float32) for _ in range(NBUF)]
    + [pltpu.SemaphoreType.DMA for _ in range(3 * NBUF)]
)


@jax.jit
def _embed_call(x2d, wt, pos):
    @pl.kernel(
        out_type=jax.ShapeDtypeStruct((BATCH * SEQ, EMB), jnp.float32),
        mesh=_mesh,
        scratch_types=_scratch,
    )
    def _embed(x_hbm, wt_hbm, pos_hbm, out_hbm, pos_sh, idx2d, *scr):
        bufs = scr[0:NBUF]
        sem_init = scr[NBUF:2 * NBUF]
        sem_g = scr[2 * NBUF:3 * NBUF]
        sem_wb = scr[3 * NBUF:4 * NBUF]

        sid = lax.axis_index("s")
        wid = sid * 2 + lax.axis_index("c")
        base = wid * TOK_PER_W

        @pl.when(sid == 0)
        def _load_pos():
            pltpu.sync_copy(pos_hbm, pos_sh)

        plsc.subcore_barrier()

        pltpu.sync_copy(x_hbm.at[pl.ds(wid * N, N)], idx2d)

        def init_descs(v, b):
            return [
                pltpu.make_async_copy(
                    pos_sh.at[pl.ds(po, n)],
                    bufs[b].at[pl.ds(bo, n)],
                    sem_init[b])
                for po, bo, n in _POS_PLAN[v]
            ]

        def d_gat(g, b):
            return pltpu.make_async_copy(
                wt_hbm.at[idx2d.at[g]], bufs[b], sem_g[b])

        def d_wb(g, b):
            return pltpu.make_async_copy(
                bufs[b], out_hbm.at[pl.ds(base + g * CHUNK, CHUNK)],
                sem_wb[b])

        def issue_pre(v, b):
            for d in init_descs(v, b):
                d.start()

        def issue_gather(g, v, b):
            for d in init_descs(v, b):
                d.wait()
            d_gat(g, b).start(add=True)

        def issue_wb(g, b):
            d_gat(g, b).wait()
            d_wb(g, b).start()

        def body(i, phase, drain):
            b0 = phase % NBUF
            b2 = (phase + 2) % NBUF
            if drain:
                d_wb(i - (NBUF - 2), b2).wait()
            issue_pre(b2, b2)
            issue_wb(i, b0)
            issue_gather(i + 2, b2, b2)

        for g in range(2):
            issue_pre(g, g)
            issue_gather(g, g, g)
        for i in range(NBUF - 2):
            body(i, i, drain=False)

        @pl.loop(NBUF - 2, N - 2, step=NBUF)
        def _steady(i0):
            for db in range(NBUF):
                body(i0 + db, NBUF - 2 + db, drain=True)

        for g in range(N - 2, N):
            issue_wb(g, g % NBUF)
        for g in range(N - NBUF, N):
            d_wb(g, g % NBUF).wait()

    return _embed(x2d, wt, pos)


def kernel(X, word_table, pos_table):
    out = _embed_call(X.reshape(-1, CHUNK), word_table, pos_table)
    return out.reshape(BATCH, SEQ, EMB)


# speedup vs baseline: 19.1277x; 1.0191x over previous
import jax
import jax.numpy as jnp
from jax import lax
from jax.experimental import pallas as pl
from jax.experimental.pallas import tpu as pltpu
from jax.experimental.pallas import tpu_sc as plsc

VOCAB = 100000
MAX_LEN = 200
EMB = 128
BATCH = 4096
SEQ = 200

NUM_WORKERS = 32
TOK_PER_W = BATCH * SEQ // NUM_WORKERS
CHUNK = 80
N = TOK_PER_W // CHUNK
NBUF = 5

_POS_PLAN = {
    0: [(0, 0, 80)],
    1: [(80, 0, 80)],
    2: [(160, 0, 40), (0, 40, 40)],
    3: [(40, 0, 80)],
    4: [(120, 0, 80)],
}

_mesh = plsc.VectorSubcoreMesh(core_axis_name="c", subcore_axis_name="s")

_scratch = (
    [pltpu.VMEM_SHARED((MAX_LEN, EMB), jnp.float32)]
    + [pltpu.VMEM((N, CHUNK), jnp.int32)]
    + [pltpu.VMEM((CHUNK, EMB), jnp.float32) for _ in range(NBUF)]
    + [pltpu.SemaphoreType.DMA for _ in range(3 * NBUF)]
)


@jax.jit
def _embed_call(x2d, wt, pos):
    @pl.kernel(
        out_type=jax.ShapeDtypeStruct((BATCH * SEQ, EMB), jnp.float32),
        mesh=_mesh,
        scratch_types=_scratch,
    )
    def _embed(x_hbm, wt_hbm, pos_hbm, out_hbm, pos_sh, idx2d, *scr):
        bufs = scr[0:NBUF]
        sem_init = scr[NBUF:2 * NBUF]
        sem_g = scr[2 * NBUF:3 * NBUF]
        sem_wb = scr[3 * NBUF:4 * NBUF]

        sid = lax.axis_index("s")
        wid = sid * 2 + lax.axis_index("c")
        base = wid * TOK_PER_W

        @pl.when(sid == 0)
        def _load_pos():
            pltpu.sync_copy(pos_hbm, pos_sh)

        plsc.subcore_barrier()

        pltpu.sync_copy(x_hbm.at[pl.ds(wid * N, N)], idx2d)

        def init_descs(v, b):
            return [
                pltpu.make_async_copy(
                    pos_sh.at[pl.ds(po, n)],
                    bufs[b].at[pl.ds(bo, n)],
                    sem_init[b])
                for po, bo, n in _POS_PLAN[v]
            ]

        def d_gat(g, b):
            return pltpu.make_async_copy(
                wt_hbm.at[idx2d.at[g]], bufs[b], sem_g[b])

        def d_wb(g, b):
            return pltpu.make_async_copy(
                bufs[b], out_hbm.at[pl.ds(base + g * CHUNK, CHUNK)],
                sem_wb[b])

        def issue_pre(v, b):
            for d in init_descs(v, b):
                d.start()

        def issue_gather(g, v, b):
            for d in init_descs(v, b):
                d.wait()
            d_gat(g, b).start(add=True)

        def issue_wb(g, b):
            d_gat(g, b).wait()
            d_wb(g, b).start()

        def body(i, phase, drain):
            b0 = phase % NBUF
            b3 = (phase + 3) % NBUF
            if drain:
                d_wb(i - (NBUF - 3), b3).wait()
            issue_pre(b3, b3)
            issue_wb(i, b0)
            issue_gather(i + 3, b3, b3)

        for g in range(3):
            issue_pre(g, g)
            issue_gather(g, g, g)
        for i in range(NBUF - 3):
            body(i, i, drain=False)

        @pl.loop(NBUF - 3, N - 3, step=NBUF)
        def _steady(i0):
            for db in range(NBUF):
                body(i0 + db, NBUF - 3 + db, drain=True)

        for g in range(N - 3, N):
            issue_wb(g, g % NBUF)
        for g in range(N - NBUF, N):
            d_wb(g, g % NBUF).wait()

    return _embed(x2d, wt, pos)


def kernel(X, word_table, pos_table):
    out = _embed_call(X.reshape(-1, CHUNK), word_table, pos_table)
    return out.reshape(BATCH, SEQ, EMB)
